# baseline (device time: 33848 ns/iter reference)
import jax
import jax.numpy as jnp
from jax import lax
from jax.experimental import pallas as pl
from jax.experimental.pallas import tpu as pltpu

M = 2048
D = 1024
HALF = M // 2
XHALF = HALF // 2
SIZES = [16, 16, 32, 48, 48, 48, 48, 48, 48, 48, 48, 32, 16, 16]
OFFS = [sum(SIZES[:i]) for i in range(len(SIZES))]
C = len(SIZES)
assert sum(SIZES) == XHALF


def kernel(partial, gamma):

    def body(p_ref, g_ref, out_ref, zbuf, localbuf, gbuf,
             z_send, z_recv, x_send, x_recv, l_sem, g_sem):
        my_x = lax.axis_index("x")
        my_y = lax.axis_index("y")
        my_z = lax.axis_index("z")
        peer_z = (my_x, my_y, 1 - my_z)
        peer_x = (1 - my_x, my_y, my_z)

        my_base = my_z * HALF + my_x * XHALF
        zp_base = (1 - my_z) * HALF + my_x * XHALF

        l_dma = pltpu.make_async_copy(
            p_ref.at[0, pl.ds(my_base, XHALF), :], localbuf, l_sem
        )
        l_dma.start()
        g_dma = pltpu.make_async_copy(g_ref, gbuf, g_sem)
        g_dma.start()

        barrier_sem = pltpu.get_barrier_semaphore()
        for peer in (peer_z, peer_x):
            pl.semaphore_signal(
                barrier_sem,
                inc=1,
                device_id=peer,
                device_id_type=pl.DeviceIdType.MESH,
            )
        pl.semaphore_wait(barrier_sem, 2)

        z_rdma = []
        for k in range(C):
            r = pltpu.make_async_remote_copy(
                src_ref=p_ref.at[0, pl.ds(zp_base + OFFS[k], SIZES[k]), :],
                dst_ref=zbuf.at[pl.ds(OFFS[k], SIZES[k]), :],
                send_sem=z_send.at[k],
                recv_sem=z_recv.at[k],
                device_id=peer_z,
                device_id_type=pl.DeviceIdType.MESH,
            )
            r.start()
            z_rdma.append(r)

        g_dma.wait()
        g = gbuf[...].reshape(1, D)
        l_dma.wait()

        x_rdma = []
        for k in range(C):
            z_rdma[k].wait_recv()
            rows = pl.ds(my_x * XHALF + OFFS[k], SIZES[k])
            local = localbuf[pl.ds(OFFS[k], SIZES[k]), :]
            y = local + zbuf[pl.ds(OFFS[k], SIZES[k]), :]
            ssq = jnp.sum(y * y, axis=-1, keepdims=True)
            inv_rms = lax.rsqrt(ssq * (1.0 / D) + 1e-6)
            out_ref[rows, :] = (y * inv_rms * g).astype(jnp.float32)
            r = pltpu.make_async_remote_copy(
                src_ref=out_ref.at[rows, :],
                dst_ref=out_ref.at[rows, :],
                send_sem=x_send.at[k],
                recv_sem=x_recv.at[k],
                device_id=peer_x,
                device_id_type=pl.DeviceIdType.MESH,
            )
            r.start()
            x_rdma.append(r)

        for k in range(C):
            z_rdma[k].wait_send()
            x_rdma[k].wait()

    return pl.pallas_call(
        body,
        out_shape=jax.ShapeDtypeStruct((HALF, D), jnp.float32),
        in_specs=[
            pl.BlockSpec(memory_space=pltpu.MemorySpace.HBM),
            pl.BlockSpec(memory_space=pltpu.MemorySpace.HBM),
        ],
        out_specs=pl.BlockSpec(memory_space=pltpu.VMEM),
        scratch_shapes=[
            pltpu.VMEM((XHALF, D), jnp.float32),
            pltpu.VMEM((XHALF, D), jnp.float32),
            pltpu.VMEM((D,), jnp.float32),
            pltpu.SemaphoreType.DMA((C,)),
            pltpu.SemaphoreType.DMA((C,)),
            pltpu.SemaphoreType.DMA((C,)),
            pltpu.SemaphoreType.DMA((C,)),
            pltpu.SemaphoreType.DMA,
            pltpu.SemaphoreType.DMA,
        ],
        compiler_params=pltpu.CompilerParams(collective_id=0),
    )(
        pltpu.with_memory_space_constraint(partial, pltpu.MemorySpace.HBM),
        pltpu.with_memory_space_constraint(gamma, pltpu.MemorySpace.HBM),
    )


# device time: 33224 ns/iter; 1.0188x vs baseline; 1.0188x over previous
import jax
import jax.numpy as jnp
from jax import lax
from jax.experimental import pallas as pl
from jax.experimental.pallas import tpu as pltpu

M = 2048
D = 1024
HALF = M // 2
XHALF = HALF // 2
C = 16
RPC = XHALF // C


def kernel(partial, gamma):

    def body(p_ref, g_ref, out_ref, zbuf, localbuf, gbuf,
             z_send, z_recv, x_send, x_recv, l_sem, g_sem):
        my_x = lax.axis_index("x")
        my_y = lax.axis_index("y")
        my_z = lax.axis_index("z")
        peer_z = (my_x, my_y, 1 - my_z)
        peer_x = (1 - my_x, my_y, my_z)

        my_base = my_z * HALF + my_x * XHALF
        zp_base = (1 - my_z) * HALF + my_x * XHALF

        l_dma = pltpu.make_async_copy(
            p_ref.at[0, pl.ds(my_base, XHALF), :], localbuf, l_sem
        )
        l_dma.start()
        g_dma = pltpu.make_async_copy(g_ref, gbuf, g_sem)
        g_dma.start()

        barrier_sem = pltpu.get_barrier_semaphore()
        for peer in (peer_z, peer_x):
            pl.semaphore_signal(
                barrier_sem,
                inc=1,
                device_id=peer,
                device_id_type=pl.DeviceIdType.MESH,
            )
        pl.semaphore_wait(barrier_sem, 2)

        z_rdma = []
        for k in range(C):
            r = pltpu.make_async_remote_copy(
                src_ref=p_ref.at[0, pl.ds(zp_base + k * RPC, RPC), :],
                dst_ref=zbuf.at[pl.ds(k * RPC, RPC), :],
                send_sem=z_send.at[k],
                recv_sem=z_recv.at[k],
                device_id=peer_z,
                device_id_type=pl.DeviceIdType.MESH,
            )
            r.start()
            z_rdma.append(r)

        g_dma.wait()
        g = gbuf[...].reshape(1, D)
        l_dma.wait()

        x_rdma = []
        for k in range(C):
            z_rdma[k].wait_recv()
            rows = pl.ds(my_x * XHALF + k * RPC, RPC)
            local = localbuf[pl.ds(k * RPC, RPC), :]
            y = local + zbuf[pl.ds(k * RPC, RPC), :]
            ssq = jnp.sum(y * y, axis=-1, keepdims=True)
            inv_rms = lax.rsqrt(ssq * (1.0 / D) + 1e-6)
            out_ref[rows, :] = (y * inv_rms * g).astype(jnp.float32)
            r = pltpu.make_async_remote_copy(
                src_ref=out_ref.at[rows, :],
                dst_ref=out_ref.at[rows, :],
                send_sem=x_send.at[k],
                recv_sem=x_recv.at[k],
                device_id=peer_x,
                device_id_type=pl.DeviceIdType.MESH,
            )
            r.start()
            x_rdma.append(r)

        for k in range(C):
            z_rdma[k].wait_send()
            x_rdma[k].wait()

    return pl.pallas_call(
        body,
        out_shape=jax.ShapeDtypeStruct((HALF, D), jnp.float32),
        in_specs=[
            pl.BlockSpec(memory_space=pltpu.MemorySpace.HBM),
            pl.BlockSpec(memory_space=pltpu.MemorySpace.HBM),
        ],
        out_specs=pl.BlockSpec(memory_space=pltpu.VMEM),
        scratch_shapes=[
            pltpu.VMEM((XHALF, D), jnp.float32),
            pltpu.VMEM((XHALF, D), jnp.float32),
            pltpu.VMEM((D,), jnp.float32),
            pltpu.SemaphoreType.DMA((C,)),
            pltpu.SemaphoreType.DMA((C,)),
            pltpu.SemaphoreType.DMA((C,)),
            pltpu.SemaphoreType.DMA((C,)),
            pltpu.SemaphoreType.DMA,
            pltpu.SemaphoreType.DMA,
        ],
        compiler_params=pltpu.CompilerParams(collective_id=0),
    )(
        pltpu.with_memory_space_constraint(partial, pltpu.MemorySpace.HBM),
        pltpu.with_memory_space_constraint(gamma, pltpu.MemorySpace.HBM),
    )
